# baseline (speedup 1.0000x reference)
import functools

import jax
import jax.numpy as jnp
from jax import lax
from jax.experimental import pallas as pl
from jax.experimental.pallas import tpu as pltpu
from jax.experimental.pallas import tpu_sc as plsc

NUM_Q = 15000
NUM_S = 200
EMB = 128
B = 128
T = 100
RANK_K = 10
QBLK = 3000
NQB = NUM_Q // QBLK
NEG = -1e30

_SC_CORES = 2
_SC_SUBCORES = 16
_NW = _SC_CORES * _SC_SUBCORES


def _dot_t(a, w):
    return lax.dot_general(a, w, (((1,), (1,)), ((), ())),
                           preferred_element_type=jnp.float32)


def _dot(a, w):
    return lax.dot_general(a, w, (((1,), (0,)), ((), ())),
                           preferred_element_type=jnp.float32)


def _ka_body(eq_ref, qn_ref, qs_ref, sn_ref, es_ref, w0_ref, b0_ref,
             f0_ref, qsp_ref, mq_ref, mf0_ref):
    step = pl.program_id(0)
    eq = eq_ref[...]
    qn = qn_ref[...]
    ios = lax.broadcasted_iota(jnp.int32, (QBLK, NUM_S), 1)
    p = ((qn[:, 0:1] == ios).astype(jnp.float32)
         + (qn[:, 1:2] == ios).astype(jnp.float32)
         + (qn[:, 2:3] == ios).astype(jnp.float32)
         + (qn[:, 3:4] == ios).astype(jnp.float32)) * 0.25
    f0 = jnp.tanh(_dot_t(_dot(p, es_ref[...]) + eq, w0_ref[...]) + b0_ref[...])
    f0_ref[...] = f0
    qsp_ref[...] = jnp.concatenate(
        [qs_ref[...], jnp.zeros((QBLK, EMB - 4), jnp.int32)], axis=1)

    sn = sn_ref[...]
    ioq = lax.broadcasted_iota(jnp.int32, (NUM_S, QBLK), 1) + step * QBLK
    c = (sn[:, 0:1] == ioq).astype(jnp.float32)
    for j in range(1, 10):
        c = c + (sn[:, j:j + 1] == ioq).astype(jnp.float32)

    @pl.when(step == 0)
    def _():
        mq_ref[...] = jnp.zeros_like(mq_ref)
        mf0_ref[...] = jnp.zeros_like(mf0_ref)

    mq_ref[...] += _dot(c, eq)
    mf0_ref[...] += _dot(c, f0)


def _kb_body(f0_ref, qn_ref, mq_ref, mf0_ref, es_ref,
             w0_ref, w1_ref, w2_ref, wl_ref, bias4_ref,
             wquery_ref, wkey_ref, watt_ref,
             agg_ref, params_ref, g0_s, g1_s):
    step = pl.program_id(0)
    b1 = bias4_ref[1:2, :]
    b2 = bias4_ref[2:3, :]
    bl = bias4_ref[3:4, :]

    @pl.when(step == 0)
    def _():
        b0 = bias4_ref[0:1, :]
        g0 = jnp.tanh(_dot_t(mq_ref[...] * 0.1 + es_ref[...], w0_ref[...]) + b0)
        g0_s[...] = g0
        g1_s[...] = jnp.tanh(_dot_t(mf0_ref[...] * 0.1 + g0, w1_ref[...]) + b1)
        watt = watt_ref[...]
        wq = _dot(watt[:, 0:EMB], wquery_ref[...])
        wk = _dot(watt[:, EMB:2 * EMB], wkey_ref[...])
        params_ref[...] = jnp.concatenate(
            [wq, wk, jnp.zeros((6, EMB), jnp.float32)], axis=0)

    qn = qn_ref[...]
    ios = lax.broadcasted_iota(jnp.int32, (QBLK, NUM_S), 1)
    p = ((qn[:, 0:1] == ios).astype(jnp.float32)
         + (qn[:, 1:2] == ios).astype(jnp.float32)
         + (qn[:, 2:3] == ios).astype(jnp.float32)
         + (qn[:, 3:4] == ios).astype(jnp.float32)) * 0.25
    f1 = jnp.tanh(_dot_t(_dot(p, g0_s[...]) + f0_ref[...], w1_ref[...]) + b1)
    f2 = jnp.tanh(_dot_t(_dot(p, g1_s[...]) + f1, w2_ref[...]) + b2)
    agg_ref[...] = jnp.tanh(_dot_t(f2, wl_ref[...]) + bl)


def _sc_gather(tables, qidx, dtypes):
    n = qidx.shape[0]
    bpw = n // _NW
    ch = 104
    nch = bpw // ch
    nt = len(tables)
    mesh = plsc.VectorSubcoreMesh(core_axis_name="c", subcore_axis_name="s")

    @functools.partial(
        pl.kernel, mesh=mesh,
        out_type=tuple(jax.ShapeDtypeStruct((n, EMB), dt) for dt in dtypes),
        scratch_types=[pltpu.VMEM((bpw,), jnp.int32)]
        + [pltpu.VMEM((ch, EMB), dt) for dt in dtypes for _ in (0, 1)]
        + [pltpu.SemaphoreType.DMA])
    def gk(*refs):
        tabs = refs[:nt]
        idx_hbm = refs[nt]
        outs = refs[nt + 1:2 * nt + 1]
        idx_v = refs[2 * nt + 1]
        bufs = refs[2 * nt + 2:2 * nt + 2 + 2 * nt]
        sem = refs[-1]
        wid = lax.axis_index("s") * _SC_CORES + lax.axis_index("c")
        base = wid * bpw
        pltpu.sync_copy(idx_hbm.at[pl.ds(base, bpw)], idx_v)
        prev = None
        for c in range(nch):
            sl = pl.ds(c * ch, ch)
            cps = [pltpu.async_copy(tabs[t].at[idx_v.at[sl]],
                                    bufs[2 * t + (c % 2)], sem)
                   for t in range(nt)]
            if prev is not None:
                pc, pcps = prev
                for cp in pcps:
                    cp.wait()
                osl = pl.ds(base + pc * ch, ch)
                for t in range(nt):
                    pltpu.sync_copy(bufs[2 * t + (pc % 2)], outs[t].at[osl])
            prev = (c, cps)
        pc, pcps = prev
        for cp in pcps:
            cp.wait()
        osl = pl.ds(base + pc * ch, ch)
        for t in range(nt):
            pltpu.sync_copy(bufs[2 * t + (pc % 2)], outs[t].at[osl])

    return gk(*tables, qidx)


BB = 4
TP = 104
SB = BB * TP


def _kp_body(eq_ref, agg_ref, qs_ref, mf_ref, rf_ref, embr_ref, wih_ref,
             bias2_ref, esp_ref, params_ref, y_ref, st_ref):
    f32 = jnp.float32
    eqf = eq_ref[...].reshape(SB, EMB)
    aggf = agg_ref[...].reshape(SB, EMB)
    qsf = qs_ref[...].reshape(SB, EMB)
    mcol = mf_ref[...].reshape(SB, 1)
    rcol = rf_ref[...].reshape(SB, 1)

    ex = jnp.where(mcol > 0.5, aggf, eqf)
    er = jnp.where(rcol > 0.5, embr_ref[1:2, :], embr_ref[0:1, :])
    x = jnp.concatenate([ex, er], axis=1)
    gates = _dot_t(x, wih_ref[...]) + bias2_ref[...]
    lstm_f = (jax.nn.sigmoid(gates[:, 3 * EMB:4 * EMB])
              * jnp.tanh(jax.nn.sigmoid(gates[:, 0:EMB])
                         * jnp.tanh(gates[:, 2 * EMB:3 * EMB])))

    rmod = lax.broadcasted_iota(jnp.int32, (BB, TP, 1), 1).reshape(SB, 1)
    bidx = lax.broadcasted_iota(jnp.int32, (BB, TP, 1), 0).reshape(SB, 1)
    shf = jnp.where(rmod == 0, 0.0, lstm_f)

    rio = lax.broadcasted_iota(jnp.int32, (SB, SB), 0)
    cio = lax.broadcasted_iota(jnp.int32, (SB, SB), 1)
    shm = jnp.where(jnp.logical_and(rio == cio + 1, rmod != 0), 1.0, 0.0)
    curf = _dot(shm, lstm_f)

    ioe = lax.broadcasted_iota(jnp.int32, (SB, 2 * EMB), 1)
    esp = esp_ref[...]
    rows = [eqf]
    for k in range(4):
        oh = (qsf[:, k:k + 1] == ioe).astype(f32)
        rows.append(_dot(oh, esp))

    wqr = params_ref[0:1, :]
    wkr = params_ref[1:2, :]
    ones_row = jnp.ones((1, EMB), f32)
    us = [_dot_t(r, wqr) for r in rows]
    ogcur = [_dot_t(r * curf, ones_row) for r in rows]
    vcur = _dot_t(curf, wkr)

    s_list, m_list = [], []
    for i in range(BB):
        lo, hi = i * TP, (i + 1) * TP
        sh_b = shf[lo:hi, :]
        eq_b = eqf[lo:hi, :]
        stack_b = jnp.concatenate(
            [eq_b, rows[1][lo:hi, :], rows[2][lo:hi, :], rows[3][lo:hi, :],
             rows[4][lo:hi, :], params_ref[...]], axis=0)
        m_list.append(_dot_t(stack_b, sh_b))
        s_list.append(_dot_t(eq_b, eq_b))
    s_mat = jnp.concatenate(s_list, axis=0)
    og = [jnp.concatenate([m_list[i][q * TP:(q + 1) * TP, :]
                           for i in range(BB)], axis=0) for q in range(5)]
    vrows = jnp.concatenate([m_list[i][5 * TP + 1:5 * TP + 2, :]
                             for i in range(BB)], axis=0)
    biota = lax.broadcasted_iota(jnp.int32, (SB, BB), 1)
    bind = (bidx == biota).astype(f32)
    vexp = _dot(bind, jnp.exp(vrows))

    ciotp = lax.broadcasted_iota(jnp.int32, (SB, TP), 1)
    cioft = ciotp.astype(f32)
    valid = ciotp <= rmod - 2
    sv = jnp.where(valid, s_mat, NEG)
    tk = jnp.zeros((SB, TP), f32)
    cur_s = sv
    for _ in range(RANK_K):
        m = jnp.max(cur_s, axis=1, keepdims=True)
        ismax = jnp.logical_and(cur_s >= m, cur_s > -1e29)
        idxm = jnp.min(jnp.where(ismax, cioft, 1e9), axis=1, keepdims=True)
        pick = cioft == idxm
        tk = tk + pick.astype(f32)
        cur_s = jnp.where(pick, NEG, cur_s)
    tksel = jnp.where(rmod <= RANK_K + 1, valid.astype(f32), tk)

    ones_col = jnp.ones((TP, 1), f32)
    ev = tksel * vexp
    sum_ev = _dot(ev, ones_col)
    evc = jnp.exp(vcur)
    num = jnp.zeros((SB, 1), f32)
    eu_sum = jnp.zeros((SB, 1), f32)
    for q in range(5):
        eu = jnp.exp(us[q])
        eu_sum = eu_sum + eu
        hist = _dot(ev * jax.nn.sigmoid(og[q]), ones_col)
        num = num + eu * (evc * jax.nn.sigmoid(ogcur[q]) + hist)
    z = eu_sum * (evc + sum_ev)
    y = num / z
    y = jnp.where(rmod == 0, 0.5, y)
    y_ref[...] = y.reshape(BB, TP, 1)
    for i in range(BB):
        st_ref[i] = lstm_f[i * TP + T - 2:i * TP + T - 1, :]


def kernel(question, response, mask, q_neighbors, s_neighbors, qs_indices,
           emb_q, emb_s, emb_r, W_ih, b_ih, W_hh, b_hh,
           W_agg0, b_agg0, W_agg1, b_agg1, W_agg2, b_agg2,
           W_agg_last, b_agg_last, W_query, b_query, W_key, b_key,
           W_att, b_att):
    f32 = jnp.float32

    zpad = jnp.zeros((B, TP - T), jnp.int32)
    qflat = jnp.concatenate([question, zpad], axis=1).reshape(-1)
    (eq_all,) = _sc_gather((emb_q,), qflat, (f32,))

    f0, qs_pad, mq, mf0 = pl.pallas_call(
        _ka_body,
        grid=(NQB,),
        in_specs=[
            pl.BlockSpec((QBLK, EMB), lambda i: (i, 0)),
            pl.BlockSpec((QBLK, 4), lambda i: (i, 0)),
            pl.BlockSpec((QBLK, 4), lambda i: (i, 0)),
            pl.BlockSpec((NUM_S, 10), lambda i: (0, 0)),
            pl.BlockSpec((NUM_S, EMB), lambda i: (0, 0)),
            pl.BlockSpec((EMB, EMB), lambda i: (0, 0)),
            pl.BlockSpec((1, EMB), lambda i: (0, 0)),
        ],
        out_specs=[
            pl.BlockSpec((QBLK, EMB), lambda i: (i, 0)),
            pl.BlockSpec((QBLK, EMB), lambda i: (i, 0)),
            pl.BlockSpec((NUM_S, EMB), lambda i: (0, 0)),
            pl.BlockSpec((NUM_S, EMB), lambda i: (0, 0)),
        ],
        out_shape=[
            jax.ShapeDtypeStruct((NUM_Q, EMB), f32),
            jax.ShapeDtypeStruct((NUM_Q, EMB), jnp.int32),
            jax.ShapeDtypeStruct((NUM_S, EMB), f32),
            jax.ShapeDtypeStruct((NUM_S, EMB), f32),
        ],
        compiler_params=pltpu.CompilerParams(
            dimension_semantics=("arbitrary",)),
    )(emb_q, q_neighbors, qs_indices, s_neighbors, emb_s, W_agg0,
      b_agg0.reshape(1, EMB))

    bias4 = jnp.stack([b_agg0, b_agg1, b_agg2, b_agg_last], axis=0)
    agg, params = pl.pallas_call(
        _kb_body,
        grid=(NQB,),
        in_specs=[
            pl.BlockSpec((QBLK, EMB), lambda i: (i, 0)),
            pl.BlockSpec((QBLK, 4), lambda i: (i, 0)),
            pl.BlockSpec((NUM_S, EMB), lambda i: (0, 0)),
            pl.BlockSpec((NUM_S, EMB), lambda i: (0, 0)),
            pl.BlockSpec((NUM_S, EMB), lambda i: (0, 0)),
            pl.BlockSpec((EMB, EMB), lambda i: (0, 0)),
            pl.BlockSpec((EMB, EMB), lambda i: (0, 0)),
            pl.BlockSpec((EMB, EMB), lambda i: (0, 0)),
            pl.BlockSpec((EMB, EMB), lambda i: (0, 0)),
            pl.BlockSpec((4, EMB), lambda i: (0, 0)),
            pl.BlockSpec((EMB, EMB), lambda i: (0, 0)),
            pl.BlockSpec((EMB, EMB), lambda i: (0, 0)),
            pl.BlockSpec((1, 2 * EMB), lambda i: (0, 0)),
        ],
        out_specs=[
            pl.BlockSpec((QBLK, EMB), lambda i: (i, 0)),
            pl.BlockSpec((8, EMB), lambda i: (0, 0)),
        ],
        out_shape=[
            jax.ShapeDtypeStruct((NUM_Q, EMB), f32),
            jax.ShapeDtypeStruct((8, EMB), f32),
        ],
        scratch_shapes=[
            pltpu.VMEM((NUM_S, EMB), f32),
            pltpu.VMEM((NUM_S, EMB), f32),
        ],
        compiler_params=pltpu.CompilerParams(
            dimension_semantics=("arbitrary",)),
    )(f0, q_neighbors, mq, mf0, emb_s, W_agg0, W_agg1, W_agg2, W_agg_last,
      bias4, W_query, W_key, W_att)

    agg_all, qs_rows = _sc_gather((agg, qs_pad), qflat,
                                  (f32, jnp.int32))

    eq3 = eq_all.reshape(B, TP, EMB)
    agg3 = agg_all.reshape(B, TP, EMB)
    qs3 = qs_rows.reshape(B, TP, EMB)
    maskf = jnp.concatenate([mask, zpad], axis=1).astype(f32).reshape(B, TP, 1)
    respf = jnp.concatenate(
        [response, zpad], axis=1).astype(f32).reshape(B, TP, 1)
    bias2 = (b_ih + b_hh).reshape(1, 4 * EMB)
    emb_s_pad = jnp.concatenate(
        [emb_s, jnp.zeros((2 * EMB - NUM_S, EMB), f32)], axis=0)

    y3, st3 = pl.pallas_call(
        _kp_body,
        grid=(B // BB,),
        in_specs=[
            pl.BlockSpec((BB, TP, EMB), lambda b: (b, 0, 0)),
            pl.BlockSpec((BB, TP, EMB), lambda b: (b, 0, 0)),
            pl.BlockSpec((BB, TP, EMB), lambda b: (b, 0, 0)),
            pl.BlockSpec((BB, TP, 1), lambda b: (b, 0, 0)),
            pl.BlockSpec((BB, TP, 1), lambda b: (b, 0, 0)),
            pl.BlockSpec((2, EMB), lambda b: (0, 0)),
            pl.BlockSpec((4 * EMB, 2 * EMB), lambda b: (0, 0)),
            pl.BlockSpec((1, 4 * EMB), lambda b: (0, 0)),
            pl.BlockSpec((2 * EMB, EMB), lambda b: (0, 0)),
            pl.BlockSpec((8, EMB), lambda b: (0, 0)),
        ],
        out_specs=[
            pl.BlockSpec((BB, TP, 1), lambda b: (b, 0, 0)),
            pl.BlockSpec((BB, 1, EMB), lambda b: (b, 0, 0)),
        ],
        out_shape=[
            jax.ShapeDtypeStruct((B, TP, 1), f32),
            jax.ShapeDtypeStruct((B, 1, EMB), f32),
        ],
        compiler_params=pltpu.CompilerParams(
            dimension_semantics=("parallel",)),
    )(eq3, agg3, qs3, maskf, respf, emb_r, W_ih, bias2, emb_s_pad, params)

    return y3.reshape(B, TP)[:, :T], st3.reshape(B, EMB)


# speedup vs baseline: 1.1297x; 1.1297x over previous
import functools

import jax
import jax.numpy as jnp
from jax import lax
from jax.experimental import pallas as pl
from jax.experimental.pallas import tpu as pltpu
from jax.experimental.pallas import tpu_sc as plsc

NUM_Q = 15000
NUM_S = 200
EMB = 128
B = 128
T = 100
RANK_K = 10
QBLK = 3000
NQB = NUM_Q // QBLK
NEG = -1e30

_SC_CORES = 2
_SC_SUBCORES = 16
_NW = _SC_CORES * _SC_SUBCORES


def _dot_t(a, w):
    return lax.dot_general(a, w, (((1,), (1,)), ((), ())),
                           preferred_element_type=jnp.float32)


def _dot(a, w):
    return lax.dot_general(a, w, (((1,), (0,)), ((), ())),
                           preferred_element_type=jnp.float32)


def _ka_body(eq_ref, qn_ref, qs_ref, sn_ref, es_ref, w0_ref, b0_ref,
             f0_ref, qsp_ref, mq_ref, mf0_ref):
    step = pl.program_id(0)
    eq = eq_ref[...]
    qn = qn_ref[...]
    ios = lax.broadcasted_iota(jnp.int32, (QBLK, NUM_S), 1)
    p = ((qn[:, 0:1] == ios).astype(jnp.float32)
         + (qn[:, 1:2] == ios).astype(jnp.float32)
         + (qn[:, 2:3] == ios).astype(jnp.float32)
         + (qn[:, 3:4] == ios).astype(jnp.float32)) * 0.25
    f0 = jnp.tanh(_dot_t(_dot(p, es_ref[...]) + eq, w0_ref[...]) + b0_ref[...])
    f0_ref[...] = f0
    qsp_ref[...] = jnp.concatenate(
        [qs_ref[...], jnp.zeros((QBLK, EMB - 4), jnp.int32)], axis=1)

    sn = sn_ref[...]
    ioq = lax.broadcasted_iota(jnp.int32, (NUM_S, QBLK), 1) + step * QBLK
    c = (sn[:, 0:1] == ioq).astype(jnp.float32)
    for j in range(1, 10):
        c = c + (sn[:, j:j + 1] == ioq).astype(jnp.float32)

    @pl.when(step == 0)
    def _():
        mq_ref[...] = jnp.zeros_like(mq_ref)
        mf0_ref[...] = jnp.zeros_like(mf0_ref)

    mq_ref[...] += _dot(c, eq)
    mf0_ref[...] += _dot(c, f0)


def _kb_body(f0_ref, qn_ref, mq_ref, mf0_ref, es_ref,
             w0_ref, w1_ref, w2_ref, wl_ref, bias4_ref,
             wquery_ref, wkey_ref, watt_ref,
             agg_ref, params_ref, g0_s, g1_s):
    step = pl.program_id(0)
    b1 = bias4_ref[1:2, :]
    b2 = bias4_ref[2:3, :]
    bl = bias4_ref[3:4, :]

    @pl.when(step == 0)
    def _():
        b0 = bias4_ref[0:1, :]
        g0 = jnp.tanh(_dot_t(mq_ref[...] * 0.1 + es_ref[...], w0_ref[...]) + b0)
        g0_s[...] = g0
        g1_s[...] = jnp.tanh(_dot_t(mf0_ref[...] * 0.1 + g0, w1_ref[...]) + b1)
        watt = watt_ref[...]
        wq = _dot(watt[:, 0:EMB], wquery_ref[...])
        wk = _dot(watt[:, EMB:2 * EMB], wkey_ref[...])
        params_ref[...] = jnp.concatenate(
            [wq, wk, jnp.zeros((6, EMB), jnp.float32)], axis=0)

    qn = qn_ref[...]
    ios = lax.broadcasted_iota(jnp.int32, (QBLK, NUM_S), 1)
    p = ((qn[:, 0:1] == ios).astype(jnp.float32)
         + (qn[:, 1:2] == ios).astype(jnp.float32)
         + (qn[:, 2:3] == ios).astype(jnp.float32)
         + (qn[:, 3:4] == ios).astype(jnp.float32)) * 0.25
    f1 = jnp.tanh(_dot_t(_dot(p, g0_s[...]) + f0_ref[...], w1_ref[...]) + b1)
    f2 = jnp.tanh(_dot_t(_dot(p, g1_s[...]) + f1, w2_ref[...]) + b2)
    agg_ref[...] = jnp.tanh(_dot_t(f2, wl_ref[...]) + bl)


def _sc_gather(tables, qidx, dtypes):
    n = qidx.shape[0]
    bpw = n // _NW
    ch = 104
    nch = bpw // ch
    nt = len(tables)
    mesh = plsc.VectorSubcoreMesh(core_axis_name="c", subcore_axis_name="s")

    @functools.partial(
        pl.kernel, mesh=mesh,
        out_type=tuple(jax.ShapeDtypeStruct((n, EMB), dt) for dt in dtypes),
        scratch_types=[pltpu.VMEM((bpw,), jnp.int32)]
        + [pltpu.VMEM((ch, EMB), dt) for dt in dtypes for _ in (0, 1)]
        + [pltpu.SemaphoreType.DMA])
    def gk(*refs):
        tabs = refs[:nt]
        idx_hbm = refs[nt]
        outs = refs[nt + 1:2 * nt + 1]
        idx_v = refs[2 * nt + 1]
        bufs = refs[2 * nt + 2:2 * nt + 2 + 2 * nt]
        sem = refs[-1]
        wid = lax.axis_index("s") * _SC_CORES + lax.axis_index("c")
        base = wid * bpw
        pltpu.sync_copy(idx_hbm.at[pl.ds(base, bpw)], idx_v)
        prev = None
        for c in range(nch):
            sl = pl.ds(c * ch, ch)
            cps = [pltpu.async_copy(tabs[t].at[idx_v.at[sl]],
                                    bufs[2 * t + (c % 2)], sem)
                   for t in range(nt)]
            if prev is not None:
                pc, pcps = prev
                for cp in pcps:
                    cp.wait()
                osl = pl.ds(base + pc * ch, ch)
                for t in range(nt):
                    pltpu.sync_copy(bufs[2 * t + (pc % 2)], outs[t].at[osl])
            prev = (c, cps)
        pc, pcps = prev
        for cp in pcps:
            cp.wait()
        osl = pl.ds(base + pc * ch, ch)
        for t in range(nt):
            pltpu.sync_copy(bufs[2 * t + (pc % 2)], outs[t].at[osl])

    return gk(*tables, qidx)


BB = 4
TP = 104
SB = BB * TP


def _kp_body(eq_ref, agg_ref, qs_ref, mf_ref, rf_ref, embr_ref, wih_ref,
             bias2_ref, esp_ref, params_ref, y_ref, st_ref):
    f32 = jnp.float32
    eqf = eq_ref[...].reshape(SB, EMB)
    aggf = agg_ref[...].reshape(SB, EMB)
    qsf = qs_ref[...].reshape(SB, EMB)
    mcol = mf_ref[...].reshape(SB, 1)
    rcol = rf_ref[...].reshape(SB, 1)

    ex = jnp.where(mcol > 0.5, aggf, eqf)
    er = jnp.where(rcol > 0.5, embr_ref[1:2, :], embr_ref[0:1, :])
    x = jnp.concatenate([ex, er], axis=1)
    gates = _dot_t(x, wih_ref[...]) + bias2_ref[...]
    lstm_f = (jax.nn.sigmoid(gates[:, 3 * EMB:4 * EMB])
              * jnp.tanh(jax.nn.sigmoid(gates[:, 0:EMB])
                         * jnp.tanh(gates[:, 2 * EMB:3 * EMB])))

    rmod = lax.broadcasted_iota(jnp.int32, (BB, TP, 1), 1).reshape(SB, 1)
    bidx = lax.broadcasted_iota(jnp.int32, (BB, TP, 1), 0).reshape(SB, 1)
    shf = jnp.where(rmod == 0, 0.0, lstm_f)

    rio = lax.broadcasted_iota(jnp.int32, (SB, SB), 0)
    cio = lax.broadcasted_iota(jnp.int32, (SB, SB), 1)
    shm = jnp.where(jnp.logical_and(rio == cio + 1, rmod != 0), 1.0, 0.0)
    curf = _dot(shm, lstm_f)

    ioe = lax.broadcasted_iota(jnp.int32, (SB, 2 * EMB), 1)
    esp = esp_ref[...]
    rows = [eqf]
    for k in range(4):
        oh = (qsf[:, k:k + 1] == ioe).astype(f32)
        rows.append(_dot(oh, esp))

    wqr = params_ref[0:1, :]
    wkr = params_ref[1:2, :]
    ones_row = jnp.ones((1, EMB), f32)
    us = [_dot_t(r, wqr) for r in rows]
    ogcur = [_dot_t(r * curf, ones_row) for r in rows]
    vcur = _dot_t(curf, wkr)

    s_list, m_list = [], []
    for i in range(BB):
        lo, hi = i * TP, (i + 1) * TP
        sh_b = shf[lo:hi, :]
        eq_b = eqf[lo:hi, :]
        stack_b = jnp.concatenate(
            [eq_b, rows[1][lo:hi, :], rows[2][lo:hi, :], rows[3][lo:hi, :],
             rows[4][lo:hi, :], params_ref[...]], axis=0)
        m_list.append(_dot_t(stack_b, sh_b))
        s_list.append(_dot_t(eq_b, eq_b))
    s_mat = jnp.concatenate(s_list, axis=0)
    og = [jnp.concatenate([m_list[i][q * TP:(q + 1) * TP, :]
                           for i in range(BB)], axis=0) for q in range(5)]
    vrows = jnp.concatenate([m_list[i][5 * TP + 1:5 * TP + 2, :]
                             for i in range(BB)], axis=0)
    biota = lax.broadcasted_iota(jnp.int32, (SB, BB), 1)
    bind = (bidx == biota).astype(f32)
    vexp = _dot(bind, jnp.exp(vrows))

    ciotp = lax.broadcasted_iota(jnp.int32, (SB, TP), 1)
    cioft = ciotp.astype(f32)
    valid = ciotp <= rmod - 2
    sv = jnp.where(valid, s_mat, NEG)
    tk = jnp.zeros((SB, TP), f32)
    cur_s = sv
    for _ in range(RANK_K):
        m = jnp.max(cur_s, axis=1, keepdims=True)
        ismax = jnp.logical_and(cur_s >= m, cur_s > -1e29)
        idxm = jnp.min(jnp.where(ismax, cioft, 1e9), axis=1, keepdims=True)
        pick = cioft == idxm
        tk = tk + pick.astype(f32)
        cur_s = jnp.where(pick, NEG, cur_s)
    tksel = jnp.where(rmod <= RANK_K + 1, valid.astype(f32), tk)

    ones_col = jnp.ones((TP, 1), f32)
    ev = tksel * vexp
    sum_ev = _dot(ev, ones_col)
    evc = jnp.exp(vcur)
    num = jnp.zeros((SB, 1), f32)
    eu_sum = jnp.zeros((SB, 1), f32)
    for q in range(5):
        eu = jnp.exp(us[q])
        eu_sum = eu_sum + eu
        hist = _dot(ev * jax.nn.sigmoid(og[q]), ones_col)
        num = num + eu * (evc * jax.nn.sigmoid(ogcur[q]) + hist)
    z = eu_sum * (evc + sum_ev)
    y = num / z
    y = jnp.where(rmod == 0, 0.5, y)
    y_ref[...] = y.reshape(BB, TP, 1)
    for i in range(BB):
        st_ref[i] = lstm_f[i * TP + T - 2:i * TP + T - 1, :]


def kernel(question, response, mask, q_neighbors, s_neighbors, qs_indices,
           emb_q, emb_s, emb_r, W_ih, b_ih, W_hh, b_hh,
           W_agg0, b_agg0, W_agg1, b_agg1, W_agg2, b_agg2,
           W_agg_last, b_agg_last, W_query, b_query, W_key, b_key,
           W_att, b_att):
    f32 = jnp.float32

    zpad = jnp.zeros((B, TP - T), jnp.int32)
    qflat = jnp.concatenate([question, zpad], axis=1).reshape(-1)

    f0, qs_pad, mq, mf0 = pl.pallas_call(
        _ka_body,
        grid=(NQB,),
        in_specs=[
            pl.BlockSpec((QBLK, EMB), lambda i: (i, 0)),
            pl.BlockSpec((QBLK, 4), lambda i: (i, 0)),
            pl.BlockSpec((QBLK, 4), lambda i: (i, 0)),
            pl.BlockSpec((NUM_S, 10), lambda i: (0, 0)),
            pl.BlockSpec((NUM_S, EMB), lambda i: (0, 0)),
            pl.BlockSpec((EMB, EMB), lambda i: (0, 0)),
            pl.BlockSpec((1, EMB), lambda i: (0, 0)),
        ],
        out_specs=[
            pl.BlockSpec((QBLK, EMB), lambda i: (i, 0)),
            pl.BlockSpec((QBLK, EMB), lambda i: (i, 0)),
            pl.BlockSpec((NUM_S, EMB), lambda i: (0, 0)),
            pl.BlockSpec((NUM_S, EMB), lambda i: (0, 0)),
        ],
        out_shape=[
            jax.ShapeDtypeStruct((NUM_Q, EMB), f32),
            jax.ShapeDtypeStruct((NUM_Q, EMB), jnp.int32),
            jax.ShapeDtypeStruct((NUM_S, EMB), f32),
            jax.ShapeDtypeStruct((NUM_S, EMB), f32),
        ],
        compiler_params=pltpu.CompilerParams(
            dimension_semantics=("arbitrary",)),
    )(emb_q, q_neighbors, qs_indices, s_neighbors, emb_s, W_agg0,
      b_agg0.reshape(1, EMB))

    bias4 = jnp.stack([b_agg0, b_agg1, b_agg2, b_agg_last], axis=0)
    agg, params = pl.pallas_call(
        _kb_body,
        grid=(NQB,),
        in_specs=[
            pl.BlockSpec((QBLK, EMB), lambda i: (i, 0)),
            pl.BlockSpec((QBLK, 4), lambda i: (i, 0)),
            pl.BlockSpec((NUM_S, EMB), lambda i: (0, 0)),
            pl.BlockSpec((NUM_S, EMB), lambda i: (0, 0)),
            pl.BlockSpec((NUM_S, EMB), lambda i: (0, 0)),
            pl.BlockSpec((EMB, EMB), lambda i: (0, 0)),
            pl.BlockSpec((EMB, EMB), lambda i: (0, 0)),
            pl.BlockSpec((EMB, EMB), lambda i: (0, 0)),
            pl.BlockSpec((EMB, EMB), lambda i: (0, 0)),
            pl.BlockSpec((4, EMB), lambda i: (0, 0)),
            pl.BlockSpec((EMB, EMB), lambda i: (0, 0)),
            pl.BlockSpec((EMB, EMB), lambda i: (0, 0)),
            pl.BlockSpec((1, 2 * EMB), lambda i: (0, 0)),
        ],
        out_specs=[
            pl.BlockSpec((QBLK, EMB), lambda i: (i, 0)),
            pl.BlockSpec((8, EMB), lambda i: (0, 0)),
        ],
        out_shape=[
            jax.ShapeDtypeStruct((NUM_Q, EMB), f32),
            jax.ShapeDtypeStruct((8, EMB), f32),
        ],
        scratch_shapes=[
            pltpu.VMEM((NUM_S, EMB), f32),
            pltpu.VMEM((NUM_S, EMB), f32),
        ],
        compiler_params=pltpu.CompilerParams(
            dimension_semantics=("arbitrary",)),
    )(f0, q_neighbors, mq, mf0, emb_s, W_agg0, W_agg1, W_agg2, W_agg_last,
      bias4, W_query, W_key, W_att)

    eq_all, agg_all, qs_rows = _sc_gather((emb_q, agg, qs_pad), qflat,
                                          (f32, f32, jnp.int32))

    eq3 = eq_all.reshape(B, TP, EMB)
    agg3 = agg_all.reshape(B, TP, EMB)
    qs3 = qs_rows.reshape(B, TP, EMB)
    maskf = jnp.concatenate([mask, zpad], axis=1).astype(f32).reshape(B, TP, 1)
    respf = jnp.concatenate(
        [response, zpad], axis=1).astype(f32).reshape(B, TP, 1)
    bias2 = (b_ih + b_hh).reshape(1, 4 * EMB)
    emb_s_pad = jnp.concatenate(
        [emb_s, jnp.zeros((2 * EMB - NUM_S, EMB), f32)], axis=0)

    y3, st3 = pl.pallas_call(
        _kp_body,
        grid=(B // BB,),
        in_specs=[
            pl.BlockSpec((BB, TP, EMB), lambda b: (b, 0, 0)),
            pl.BlockSpec((BB, TP, EMB), lambda b: (b, 0, 0)),
            pl.BlockSpec((BB, TP, EMB), lambda b: (b, 0, 0)),
            pl.BlockSpec((BB, TP, 1), lambda b: (b, 0, 0)),
            pl.BlockSpec((BB, TP, 1), lambda b: (b, 0, 0)),
            pl.BlockSpec((2, EMB), lambda b: (0, 0)),
            pl.BlockSpec((4 * EMB, 2 * EMB), lambda b: (0, 0)),
            pl.BlockSpec((1, 4 * EMB), lambda b: (0, 0)),
            pl.BlockSpec((2 * EMB, EMB), lambda b: (0, 0)),
            pl.BlockSpec((8, EMB), lambda b: (0, 0)),
        ],
        out_specs=[
            pl.BlockSpec((BB, TP, 1), lambda b: (b, 0, 0)),
            pl.BlockSpec((BB, 1, EMB), lambda b: (b, 0, 0)),
        ],
        out_shape=[
            jax.ShapeDtypeStruct((B, TP, 1), f32),
            jax.ShapeDtypeStruct((B, 1, EMB), f32),
        ],
        compiler_params=pltpu.CompilerParams(
            dimension_semantics=("parallel",)),
    )(eq3, agg3, qs3, maskf, respf, emb_r, W_ih, bias2, emb_s_pad, params)

    return y3.reshape(B, TP)[:, :T], st3.reshape(B, EMB)


# speedup vs baseline: 1.2159x; 1.0763x over previous
import functools

import jax
import jax.numpy as jnp
from jax import lax
from jax.experimental import pallas as pl
from jax.experimental.pallas import tpu as pltpu
from jax.experimental.pallas import tpu_sc as plsc

NUM_Q = 15000
NUM_S = 200
EMB = 128
B = 128
T = 100
RANK_K = 10
QBLK = 3000
NQB = NUM_Q // QBLK
NEG = -1e30

_SC_CORES = 2
_SC_SUBCORES = 16
_NW = _SC_CORES * _SC_SUBCORES


def _dot_t(a, w):
    return lax.dot_general(a, w, (((1,), (1,)), ((), ())),
                           preferred_element_type=jnp.float32)


def _dot(a, w):
    return lax.dot_general(a, w, (((1,), (0,)), ((), ())),
                           preferred_element_type=jnp.float32)


def _ka_body(eq_ref, qn_ref, qs_ref, sn_ref, es_ref, w0_ref, b0_ref,
             f0_ref, qsp_ref, mq_ref, mf0_ref):
    step = pl.program_id(0)
    eq = eq_ref[...]
    qn = qn_ref[...]
    ios = lax.broadcasted_iota(jnp.int32, (QBLK, NUM_S), 1)
    p = ((qn[:, 0:1] == ios).astype(jnp.float32)
         + (qn[:, 1:2] == ios).astype(jnp.float32)
         + (qn[:, 2:3] == ios).astype(jnp.float32)
         + (qn[:, 3:4] == ios).astype(jnp.float32)) * 0.25
    f0 = jnp.tanh(_dot_t(_dot(p, es_ref[...]) + eq, w0_ref[...]) + b0_ref[...])
    f0_ref[...] = f0
    qsp_ref[...] = jnp.concatenate(
        [qs_ref[...], jnp.zeros((QBLK, EMB - 4), jnp.int32)], axis=1)

    sn = sn_ref[...]
    ioq = lax.broadcasted_iota(jnp.int32, (NUM_S, QBLK), 1) + step * QBLK
    c = (sn[:, 0:1] == ioq).astype(jnp.float32)
    for j in range(1, 10):
        c = c + (sn[:, j:j + 1] == ioq).astype(jnp.float32)

    @pl.when(step == 0)
    def _():
        mq_ref[...] = jnp.zeros_like(mq_ref)
        mf0_ref[...] = jnp.zeros_like(mf0_ref)

    mq_ref[...] += _dot(c, eq)
    mf0_ref[...] += _dot(c, f0)


def _kb_body(f0_ref, qn_ref, mq_ref, mf0_ref, es_ref,
             w0_ref, w1_ref, w2_ref, wl_ref, bias4_ref,
             wquery_ref, wkey_ref, watt_ref,
             agg_ref, params_ref, g0_s, g1_s):
    step = pl.program_id(0)
    b1 = bias4_ref[1:2, :]
    b2 = bias4_ref[2:3, :]
    bl = bias4_ref[3:4, :]

    @pl.when(step == 0)
    def _():
        b0 = bias4_ref[0:1, :]
        g0 = jnp.tanh(_dot_t(mq_ref[...] * 0.1 + es_ref[...], w0_ref[...]) + b0)
        g0_s[...] = g0
        g1_s[...] = jnp.tanh(_dot_t(mf0_ref[...] * 0.1 + g0, w1_ref[...]) + b1)
        watt = watt_ref[...]
        wq = _dot(watt[:, 0:EMB], wquery_ref[...])
        wk = _dot(watt[:, EMB:2 * EMB], wkey_ref[...])
        params_ref[...] = jnp.concatenate(
            [wq, wk, jnp.zeros((6, EMB), jnp.float32)], axis=0)

    qn = qn_ref[...]
    ios = lax.broadcasted_iota(jnp.int32, (QBLK, NUM_S), 1)
    p = ((qn[:, 0:1] == ios).astype(jnp.float32)
         + (qn[:, 1:2] == ios).astype(jnp.float32)
         + (qn[:, 2:3] == ios).astype(jnp.float32)
         + (qn[:, 3:4] == ios).astype(jnp.float32)) * 0.25
    f1 = jnp.tanh(_dot_t(_dot(p, g0_s[...]) + f0_ref[...], w1_ref[...]) + b1)
    f2 = jnp.tanh(_dot_t(_dot(p, g1_s[...]) + f1, w2_ref[...]) + b2)
    agg_ref[...] = jnp.tanh(_dot_t(f2, wl_ref[...]) + bl)


def _sc_gather(tables, qidx, dtypes):
    n = qidx.shape[0]
    bpw = n // _NW
    ch = 104
    nch = bpw // ch
    nt = len(tables)
    mesh = plsc.VectorSubcoreMesh(core_axis_name="c", subcore_axis_name="s")

    @functools.partial(
        pl.kernel, mesh=mesh,
        out_type=tuple(jax.ShapeDtypeStruct((n, EMB), dt) for dt in dtypes),
        scratch_types=[pltpu.VMEM((bpw,), jnp.int32)]
        + [pltpu.VMEM((ch, EMB), dt) for dt in dtypes for _ in (0, 1)]
        + [pltpu.SemaphoreType.DMA])
    def gk(*refs):
        tabs = refs[:nt]
        idx_hbm = refs[nt]
        outs = refs[nt + 1:2 * nt + 1]
        idx_v = refs[2 * nt + 1]
        bufs = refs[2 * nt + 2:2 * nt + 2 + 2 * nt]
        sem = refs[-1]
        wid = lax.axis_index("s") * _SC_CORES + lax.axis_index("c")
        base = wid * bpw
        pltpu.sync_copy(idx_hbm.at[pl.ds(base, bpw)], idx_v)
        prev = None
        for c in range(nch):
            sl = pl.ds(c * ch, ch)
            cps = [pltpu.async_copy(tabs[t].at[idx_v.at[sl]],
                                    bufs[2 * t + (c % 2)], sem)
                   for t in range(nt)]
            if prev is not None:
                pc, pcps = prev
                for cp in pcps:
                    cp.wait()
                osl = pl.ds(base + pc * ch, ch)
                for t in range(nt):
                    pltpu.sync_copy(bufs[2 * t + (pc % 2)], outs[t].at[osl])
            prev = (c, cps)
        pc, pcps = prev
        for cp in pcps:
            cp.wait()
        osl = pl.ds(base + pc * ch, ch)
        for t in range(nt):
            pltpu.sync_copy(bufs[2 * t + (pc % 2)], outs[t].at[osl])

    return gk(*tables, qidx)


BB = 8
TP = 104
SB = BB * TP


def _kp_body(eq_ref, agg_ref, qs_ref, mf_ref, rf_ref, embr_ref, wih_ref,
             bias2_ref, esp_ref, params_ref, y_ref, st_ref):
    f32 = jnp.float32
    eqf = eq_ref[...].reshape(SB, EMB)
    aggf = agg_ref[...].reshape(SB, EMB)
    qsf = qs_ref[...].reshape(SB, EMB)
    mcol = mf_ref[...].reshape(SB, 1)
    rcol = rf_ref[...].reshape(SB, 1)

    ex = jnp.where(mcol > 0.5, aggf, eqf)
    er = jnp.where(rcol > 0.5, embr_ref[1:2, :], embr_ref[0:1, :])
    x = jnp.concatenate([ex, er], axis=1)
    gates = _dot_t(x, wih_ref[...]) + bias2_ref[...]
    lstm_f = (jax.nn.sigmoid(gates[:, 3 * EMB:4 * EMB])
              * jnp.tanh(jax.nn.sigmoid(gates[:, 0:EMB])
                         * jnp.tanh(gates[:, 2 * EMB:3 * EMB])))

    rmod = lax.broadcasted_iota(jnp.int32, (BB, TP, 1), 1).reshape(SB, 1)
    bidx = lax.broadcasted_iota(jnp.int32, (BB, TP, 1), 0).reshape(SB, 1)
    shf = jnp.where(rmod == 0, 0.0, lstm_f)

    rio = lax.broadcasted_iota(jnp.int32, (SB, SB), 0)
    cio = lax.broadcasted_iota(jnp.int32, (SB, SB), 1)
    shm = jnp.where(jnp.logical_and(rio == cio + 1, rmod != 0), 1.0, 0.0)
    curf = _dot(shm, lstm_f)

    ioe = lax.broadcasted_iota(jnp.int32, (SB, 2 * EMB), 1)
    esp = esp_ref[...]
    rows = [eqf]
    for k in range(4):
        oh = (qsf[:, k:k + 1] == ioe).astype(f32)
        rows.append(_dot(oh, esp))

    wqr = params_ref[0:1, :]
    wkr = params_ref[1:2, :]
    ones_row = jnp.ones((1, EMB), f32)
    us = [_dot_t(r, wqr) for r in rows]
    ogcur = [_dot_t(r * curf, ones_row) for r in rows]
    vcur = _dot_t(curf, wkr)

    s_list, m_list = [], []
    for i in range(BB):
        lo, hi = i * TP, (i + 1) * TP
        sh_b = shf[lo:hi, :]
        eq_b = eqf[lo:hi, :]
        stack_b = jnp.concatenate(
            [eq_b, rows[1][lo:hi, :], rows[2][lo:hi, :], rows[3][lo:hi, :],
             rows[4][lo:hi, :], params_ref[...]], axis=0)
        m_list.append(_dot_t(stack_b, sh_b))
        s_list.append(_dot_t(eq_b, eq_b))
    s_mat = jnp.concatenate(s_list, axis=0)
    og = [jnp.concatenate([m_list[i][q * TP:(q + 1) * TP, :]
                           for i in range(BB)], axis=0) for q in range(5)]
    vrows = jnp.concatenate([m_list[i][5 * TP + 1:5 * TP + 2, :]
                             for i in range(BB)], axis=0)
    biota = lax.broadcasted_iota(jnp.int32, (SB, BB), 1)
    bind = (bidx == biota).astype(f32)
    vexp = _dot(bind, jnp.exp(vrows))

    ciotp = lax.broadcasted_iota(jnp.int32, (SB, TP), 1)
    cioft = ciotp.astype(f32)
    valid = ciotp <= rmod - 2
    sv = jnp.where(valid, s_mat, NEG)
    tk = jnp.zeros((SB, TP), f32)
    cur_s = sv
    for _ in range(RANK_K):
        m = jnp.max(cur_s, axis=1, keepdims=True)
        ismax = jnp.logical_and(cur_s >= m, cur_s > -1e29)
        idxm = jnp.min(jnp.where(ismax, cioft, 1e9), axis=1, keepdims=True)
        pick = cioft == idxm
        tk = tk + pick.astype(f32)
        cur_s = jnp.where(pick, NEG, cur_s)
    tksel = jnp.where(rmod <= RANK_K + 1, valid.astype(f32), tk)

    ones_col = jnp.ones((TP, 1), f32)
    ev = tksel * vexp
    sum_ev = _dot(ev, ones_col)
    evc = jnp.exp(vcur)
    num = jnp.zeros((SB, 1), f32)
    eu_sum = jnp.zeros((SB, 1), f32)
    for q in range(5):
        eu = jnp.exp(us[q])
        eu_sum = eu_sum + eu
        hist = _dot(ev * jax.nn.sigmoid(og[q]), ones_col)
        num = num + eu * (evc * jax.nn.sigmoid(ogcur[q]) + hist)
    z = eu_sum * (evc + sum_ev)
    y = num / z
    y = jnp.where(rmod == 0, 0.5, y)
    y_ref[...] = y.reshape(BB, TP, 1)
    for i in range(BB):
        st_ref[i] = lstm_f[i * TP + T - 2:i * TP + T - 1, :]


def kernel(question, response, mask, q_neighbors, s_neighbors, qs_indices,
           emb_q, emb_s, emb_r, W_ih, b_ih, W_hh, b_hh,
           W_agg0, b_agg0, W_agg1, b_agg1, W_agg2, b_agg2,
           W_agg_last, b_agg_last, W_query, b_query, W_key, b_key,
           W_att, b_att):
    f32 = jnp.float32

    zpad = jnp.zeros((B, TP - T), jnp.int32)
    qflat = jnp.concatenate([question, zpad], axis=1).reshape(-1)

    f0, qs_pad, mq, mf0 = pl.pallas_call(
        _ka_body,
        grid=(NQB,),
        in_specs=[
            pl.BlockSpec((QBLK, EMB), lambda i: (i, 0)),
            pl.BlockSpec((QBLK, 4), lambda i: (i, 0)),
            pl.BlockSpec((QBLK, 4), lambda i: (i, 0)),
            pl.BlockSpec((NUM_S, 10), lambda i: (0, 0)),
            pl.BlockSpec((NUM_S, EMB), lambda i: (0, 0)),
            pl.BlockSpec((EMB, EMB), lambda i: (0, 0)),
            pl.BlockSpec((1, EMB), lambda i: (0, 0)),
        ],
        out_specs=[
            pl.BlockSpec((QBLK, EMB), lambda i: (i, 0)),
            pl.BlockSpec((QBLK, EMB), lambda i: (i, 0)),
            pl.BlockSpec((NUM_S, EMB), lambda i: (0, 0)),
            pl.BlockSpec((NUM_S, EMB), lambda i: (0, 0)),
        ],
        out_shape=[
            jax.ShapeDtypeStruct((NUM_Q, EMB), f32),
            jax.ShapeDtypeStruct((NUM_Q, EMB), jnp.int32),
            jax.ShapeDtypeStruct((NUM_S, EMB), f32),
            jax.ShapeDtypeStruct((NUM_S, EMB), f32),
        ],
        compiler_params=pltpu.CompilerParams(
            dimension_semantics=("arbitrary",)),
    )(emb_q, q_neighbors, qs_indices, s_neighbors, emb_s, W_agg0,
      b_agg0.reshape(1, EMB))

    bias4 = jnp.stack([b_agg0, b_agg1, b_agg2, b_agg_last], axis=0)
    agg, params = pl.pallas_call(
        _kb_body,
        grid=(NQB,),
        in_specs=[
            pl.BlockSpec((QBLK, EMB), lambda i: (i, 0)),
            pl.BlockSpec((QBLK, 4), lambda i: (i, 0)),
            pl.BlockSpec((NUM_S, EMB), lambda i: (0, 0)),
            pl.BlockSpec((NUM_S, EMB), lambda i: (0, 0)),
            pl.BlockSpec((NUM_S, EMB), lambda i: (0, 0)),
            pl.BlockSpec((EMB, EMB), lambda i: (0, 0)),
            pl.BlockSpec((EMB, EMB), lambda i: (0, 0)),
            pl.BlockSpec((EMB, EMB), lambda i: (0, 0)),
            pl.BlockSpec((EMB, EMB), lambda i: (0, 0)),
            pl.BlockSpec((4, EMB), lambda i: (0, 0)),
            pl.BlockSpec((EMB, EMB), lambda i: (0, 0)),
            pl.BlockSpec((EMB, EMB), lambda i: (0, 0)),
            pl.BlockSpec((1, 2 * EMB), lambda i: (0, 0)),
        ],
        out_specs=[
            pl.BlockSpec((QBLK, EMB), lambda i: (i, 0)),
            pl.BlockSpec((8, EMB), lambda i: (0, 0)),
        ],
        out_shape=[
            jax.ShapeDtypeStruct((NUM_Q, EMB), f32),
            jax.ShapeDtypeStruct((8, EMB), f32),
        ],
        scratch_shapes=[
            pltpu.VMEM((NUM_S, EMB), f32),
            pltpu.VMEM((NUM_S, EMB), f32),
        ],
        compiler_params=pltpu.CompilerParams(
            dimension_semantics=("arbitrary",)),
    )(f0, q_neighbors, mq, mf0, emb_s, W_agg0, W_agg1, W_agg2, W_agg_last,
      bias4, W_query, W_key, W_att)

    eq_all, agg_all, qs_rows = _sc_gather((emb_q, agg, qs_pad), qflat,
                                          (f32, f32, jnp.int32))

    eq3 = eq_all.reshape(B, TP, EMB)
    agg3 = agg_all.reshape(B, TP, EMB)
    qs3 = qs_rows.reshape(B, TP, EMB)
    maskf = jnp.concatenate([mask, zpad], axis=1).astype(f32).reshape(B, TP, 1)
    respf = jnp.concatenate(
        [response, zpad], axis=1).astype(f32).reshape(B, TP, 1)
    bias2 = (b_ih + b_hh).reshape(1, 4 * EMB)
    emb_s_pad = jnp.concatenate(
        [emb_s, jnp.zeros((2 * EMB - NUM_S, EMB), f32)], axis=0)

    y3, st3 = pl.pallas_call(
        _kp_body,
        grid=(B // BB,),
        in_specs=[
            pl.BlockSpec((BB, TP, EMB), lambda b: (b, 0, 0)),
            pl.BlockSpec((BB, TP, EMB), lambda b: (b, 0, 0)),
            pl.BlockSpec((BB, TP, EMB), lambda b: (b, 0, 0)),
            pl.BlockSpec((BB, TP, 1), lambda b: (b, 0, 0)),
            pl.BlockSpec((BB, TP, 1), lambda b: (b, 0, 0)),
            pl.BlockSpec((2, EMB), lambda b: (0, 0)),
            pl.BlockSpec((4 * EMB, 2 * EMB), lambda b: (0, 0)),
            pl.BlockSpec((1, 4 * EMB), lambda b: (0, 0)),
            pl.BlockSpec((2 * EMB, EMB), lambda b: (0, 0)),
            pl.BlockSpec((8, EMB), lambda b: (0, 0)),
        ],
        out_specs=[
            pl.BlockSpec((BB, TP, 1), lambda b: (b, 0, 0)),
            pl.BlockSpec((BB, 1, EMB), lambda b: (b, 0, 0)),
        ],
        out_shape=[
            jax.ShapeDtypeStruct((B, TP, 1), f32),
            jax.ShapeDtypeStruct((B, 1, EMB), f32),
        ],
        compiler_params=pltpu.CompilerParams(
            dimension_semantics=("parallel",)),
    )(eq3, agg3, qs3, maskf, respf, emb_r, W_ih, bias2, emb_s_pad, params)

    return y3.reshape(B, TP)[:, :T], st3.reshape(B, EMB)
